# Initial kernel scaffold; baseline (speedup 1.0000x reference)
import jax
import jax.numpy as jnp
from jax.experimental import pallas as pl


def kernel(x, edge_index, W1, b1, W2, b2):
    raise NotImplementedError("write your pallas kernel here")


# speedup vs baseline: 12.3434x; 12.3434x over previous
import functools

import jax
import jax.numpy as jnp
from jax import lax
from jax.experimental import pallas as pl
from jax.experimental.pallas import tpu as pltpu
from jax.experimental.pallas import tpu_sc as plsc

N = 10000
E = 320000
D = 128

NC = 2
NS = 16
NW = NC * NS
K = 128
CH = -(-E // (NW * K))
EPAD = NW * CH * K
NPAD = CH * K
RPS = NPAD // NS

_mesh = plsc.VectorSubcoreMesh(core_axis_name="c", subcore_axis_name="s")


@functools.partial(
    pl.kernel,
    out_type=jax.ShapeDtypeStruct((NW, NPAD), jnp.float32),
    mesh=_mesh,
    scratch_types=[
        pltpu.VMEM((CH * K,), jnp.int32),
        pltpu.VMEM((NPAD,), jnp.float32),
    ],
    compiler_params=pltpu.CompilerParams(needs_layout_passes=False),
)
def _deg_kernel(dstf_hbm, zeros_hbm, deg_hbm, idx_v, acc_v):
    c = lax.axis_index("c")
    s = lax.axis_index("s")
    wid = s * NC + c
    pltpu.sync_copy(zeros_hbm, acc_v)
    pltpu.sync_copy(dstf_hbm.at[wid], idx_v)
    ones = jnp.full((16,), 1.0, jnp.float32)

    def body(i, carry):
        ids = idx_v[pl.ds(pl.multiple_of(i * 16, 16), 16)]
        plsc.addupdate_scatter(acc_v, [ids], ones)
        return carry

    lax.fori_loop(0, (CH * K) // 16, body, 0)
    pltpu.sync_copy(acc_v, deg_hbm.at[wid])


@functools.partial(
    pl.kernel,
    out_type=jax.ShapeDtypeStruct((NC, NPAD, D), jnp.float32),
    mesh=_mesh,
    scratch_types=[
        pltpu.VMEM((CH, K), jnp.int32),
        pltpu.VMEM((CH, K), jnp.int32),
        pltpu.VMEM((K, D), jnp.float32),
        pltpu.VMEM_SHARED((NPAD, D), jnp.float32),
        pltpu.SemaphoreType.DMA,
    ],
)
def _agg_kernel(y_hbm, src_hbm, dst_hbm, parts_hbm, src_v, dst_v, rows_v, acc_sh, gsem):
    c = lax.axis_index("c")
    s = lax.axis_index("s")
    wid = s * NC + c
    row0 = pl.multiple_of(s * RPS, 8)
    pltpu.sync_copy(y_hbm.at[pl.ds(row0, RPS)], acc_sh.at[pl.ds(row0, RPS)])
    pltpu.sync_copy(src_hbm.at[wid], src_v)
    pltpu.sync_copy(dst_hbm.at[wid], dst_v)
    plsc.subcore_barrier()

    def body(j, carry):
        pltpu.async_copy(y_hbm.at[src_v.at[j]], rows_v, gsem).wait()
        pltpu.sync_copy(rows_v, acc_sh.at[dst_v.at[j]], add=True)
        return carry

    lax.fori_loop(0, CH, body, 0)
    plsc.subcore_barrier()
    pltpu.sync_copy(acc_sh.at[pl.ds(row0, RPS)], parts_hbm.at[c, pl.ds(row0, RPS)])


_BLK = 128


def _tc1_body(parts_ref, x_ref, w_ref, y_ref, dis_ref):
    deg = jnp.sum(parts_ref[...], axis=0) + 1.0
    dis = lax.rsqrt(deg)
    y = jnp.dot(x_ref[...], w_ref[...], preferred_element_type=jnp.float32)
    y_ref[...] = y * dis[:, None]
    dis_ref[...] = dis


def _tc1(deg_parts, x_pad, W1):
    return pl.pallas_call(
        _tc1_body,
        grid=(NPAD // _BLK,),
        in_specs=[
            pl.BlockSpec((NW, _BLK), lambda i: (0, i)),
            pl.BlockSpec((_BLK, D), lambda i: (i, 0)),
            pl.BlockSpec((D, D), lambda i: (0, 0)),
        ],
        out_specs=[
            pl.BlockSpec((_BLK, D), lambda i: (i, 0)),
            pl.BlockSpec((_BLK,), lambda i: (i,)),
        ],
        out_shape=[
            jax.ShapeDtypeStruct((NPAD, D), jnp.float32),
            jax.ShapeDtypeStruct((NPAD,), jnp.float32),
        ],
    )(deg_parts, x_pad, W1)


def _tc2_body(parts_ref, y_ref, dis_ref, b_ref, w_ref, y2_ref):
    agg = parts_ref[0] + parts_ref[1] - y_ref[...]
    dis = dis_ref[...]
    h = jnp.maximum(agg * dis[:, None] + b_ref[...][None, :], 0.0)
    y2 = jnp.dot(h, w_ref[...], preferred_element_type=jnp.float32)
    y2_ref[...] = y2 * dis[:, None]


def _tc2(parts, y1, dis, b1, W2):
    return pl.pallas_call(
        _tc2_body,
        grid=(NPAD // _BLK,),
        in_specs=[
            pl.BlockSpec((NC, _BLK, D), lambda i: (0, i, 0)),
            pl.BlockSpec((_BLK, D), lambda i: (i, 0)),
            pl.BlockSpec((_BLK,), lambda i: (i,)),
            pl.BlockSpec((D,), lambda i: (0,)),
            pl.BlockSpec((D, D), lambda i: (0, 0)),
        ],
        out_specs=pl.BlockSpec((_BLK, D), lambda i: (i, 0)),
        out_shape=jax.ShapeDtypeStruct((NPAD, D), jnp.float32),
    )(parts, y1, dis, b1, W2)


def _tc3_body(parts_ref, y_ref, dis_ref, b_ref, o_ref):
    agg = parts_ref[0] + parts_ref[1] - y_ref[...]
    o_ref[...] = jnp.maximum(agg * dis_ref[...][:, None] + b_ref[...][None, :], 0.0)


def _tc3(parts, y2, dis, b2):
    return pl.pallas_call(
        _tc3_body,
        grid=(NPAD // _BLK,),
        in_specs=[
            pl.BlockSpec((NC, _BLK, D), lambda i: (0, i, 0)),
            pl.BlockSpec((_BLK, D), lambda i: (i, 0)),
            pl.BlockSpec((_BLK,), lambda i: (i,)),
            pl.BlockSpec((D,), lambda i: (0,)),
        ],
        out_specs=pl.BlockSpec((_BLK, D), lambda i: (i, 0)),
        out_shape=jax.ShapeDtypeStruct((NPAD, D), jnp.float32),
    )(parts, y2, dis, b2)


def kernel(x, edge_index, W1, b1, W2, b2):
    src = edge_index[0]
    dst = edge_index[1]
    pad_e = EPAD - E
    src_p = jnp.concatenate([src, jnp.zeros((pad_e,), jnp.int32)])
    dst_p = jnp.concatenate([dst, jnp.full((pad_e,), N, jnp.int32)])
    src3 = src_p.reshape(NW, CH, K)
    dst3 = dst_p.reshape(NW, CH, K)
    dst_flat = dst_p.reshape(NW, CH * K)
    x_pad = jnp.pad(x, ((0, NPAD - N), (0, 0)))
    zeros_row = jnp.zeros((NPAD,), jnp.float32)

    deg_parts = _deg_kernel(dst_flat, zeros_row)
    y1, dis = _tc1(deg_parts, x_pad, W1)
    parts1 = _agg_kernel(y1, src3, dst3)
    y2 = _tc2(parts1, y1, dis, b1, W2)
    parts2 = _agg_kernel(y2, src3, dst3)
    out = _tc3(parts2, y2, dis, b2)
    return out[:N]
